# Initial kernel scaffold; baseline (speedup 1.0000x reference)
import jax
import jax.numpy as jnp
from jax.experimental import pallas as pl


def kernel(x, edge_index, W1, b1, W2, b2, Wih_f, Whh_f, bih_f, bhh_f, Wih_b, Whh_b, bih_b, bhh_b, att_W, att_b, lin_W, lin_b):
    raise NotImplementedError("write your pallas kernel here")


# speedup vs baseline: 21.2135x; 21.2135x over previous
import functools

import jax
import jax.numpy as jnp
from jax import lax
from jax.experimental import pallas as pl
from jax.experimental.pallas import tpu as pltpu
from jax.experimental.pallas import tpu_sc as plsc

N = 10000
E = 320000
NP = 10112
EP = 327680
EPR = EP // 128
NTILES = 16
EPT = EP // NTILES
WPT = EPR // NTILES
RPT = NP // NTILES
_NB = 4


@functools.lru_cache(maxsize=None)
def _make_propagate(fh):
    mesh = plsc.VectorSubcoreMesh(core_axis_name="c", subcore_axis_name="s",
                                  num_cores=2, num_subcores=16)
    wide = fh > 32
    gw = 128 if wide else 512
    nbuf = _NB if wide else 2
    nwin = EPT // gw

    @functools.partial(
        pl.kernel,
        out_type=(
            jax.ShapeDtypeStruct((NP, fh), jnp.float32),
            jax.ShapeDtypeStruct((NP, fh), jnp.float32),
        ),
        mesh=mesh,
        scratch_types=[
            pltpu.VMEM((EPT,), jnp.int32),
            pltpu.VMEM((WPT, 128), jnp.int32),
            [pltpu.VMEM((gw, fh), jnp.float32) for _ in range(nbuf)],
            pltpu.VMEM_SHARED((NP, fh), jnp.float32),
            [pltpu.SemaphoreType.DMA for _ in range(nbuf)],
        ],
        compiler_params=pltpu.CompilerParams(use_tc_tiling_on_sc=False),
    )
    def prop(h_lo, h_hi, src1d, dst2d, zeros_hbm, out_lo, out_hi,
             src_v, dst_v, rows, acc, sems):
        c = lax.axis_index("c")
        s = lax.axis_index("s")

        e0 = pl.multiple_of(s * EPT, EPT)
        pltpu.sync_copy(src1d.at[pl.ds(e0, EPT)], src_v)
        w0 = pl.multiple_of(s * WPT, WPT)
        pltpu.sync_copy(dst2d.at[pl.ds(w0, WPT)], dst_v)

        r0 = pl.multiple_of(s * RPT, RPT)
        pltpu.sync_copy(zeros_hbm.at[pl.ds(r0, RPT)], acc.at[pl.ds(r0, RPT)])
        plsc.subcore_barrier()

        def start_gather(g, b):
            i0 = pl.multiple_of(g * gw, gw)
            idx = src_v.at[pl.ds(i0, gw)]

            @pl.when(c == 0)
            def _():
                pltpu.async_copy(h_lo.at[idx], rows[b], sems[b])

            @pl.when(c != 0)
            def _():
                pltpu.async_copy(h_hi.at[idx], rows[b], sems[b])

        def wait_gather(b):
            pltpu.make_async_copy(h_lo.at[src_v.at[pl.ds(0, gw)]],
                                  rows[b], sems[b]).wait()

        def scatter(g, b):
            for j in range(gw // 128):
                src = rows[b] if gw == 128 else rows[b].at[pl.ds(j * 128, 128)]
                pltpu.sync_copy(src, acc.at[dst_v.at[g * (gw // 128) + j]],
                                add=True)

        for b in range(nbuf):
            start_gather(b, b)

        def body(g0, carry):
            for b in range(nbuf):
                g = g0 * nbuf + b
                wait_gather(b)
                scatter(g, b)
                start_gather(g + nbuf, b)
            return carry

        lax.fori_loop(0, nwin // nbuf - 1, body, 0)
        for b in range(nbuf):
            g = (nwin // nbuf - 1) * nbuf + b
            wait_gather(b)
            scatter(g, b)
        plsc.subcore_barrier()

        @pl.when(c == 0)
        def _():
            pltpu.sync_copy(acc.at[pl.ds(r0, RPT)], out_lo.at[pl.ds(r0, RPT)])

        @pl.when(c != 0)
        def _():
            pltpu.sync_copy(acc.at[pl.ds(r0, RPT)], out_hi.at[pl.ds(r0, RPT)])

    return prop


def _propagate(fh, h_lo, h_hi, src1d, dst2d, zeros_hbm):
    return _make_propagate(fh)(h_lo, h_hi, src1d, dst2d, zeros_hbm)


def _dinv_of(deg16):
    return lax.rsqrt(deg16[:, 0:1] + 1.0)


def _row_mask(blk):
    base = pl.program_id(0) * blk
    rid = lax.broadcasted_iota(jnp.int32, (blk, 1), 0) + base
    return rid < N


def _prescale_body(x_ref, deg_ref, lo_ref, hi_ref):
    blk = lo_ref.shape[0]
    m = _row_mask(blk)
    xt = jnp.where(m, x_ref[...] * _dinv_of(deg_ref[...]), 0.0)
    lo_ref[...] = xt[:, :64]
    hi_ref[...] = xt[:, 64:]


def _layer1_body(a0_ref, a1_ref, x0_ref, x1_ref, deg_ref, w_ref, b_ref,
                 x_out_ref, q0_ref, q1_ref, q2_ref, q3_ref):
    blk = x_out_ref.shape[0]
    m = _row_mask(blk)
    dinv = jnp.where(m, _dinv_of(deg_ref[...]), 0.0)
    px = jnp.concatenate(
        [a0_ref[...] + x0_ref[...], a1_ref[...] + x1_ref[...]], axis=1) * dinv
    h = jnp.dot(px, w_ref[...], preferred_element_type=jnp.float32) + b_ref[...]
    xo = jnp.maximum(h, 0.0)
    x_out_ref[...] = xo
    xt = xo * dinv
    q0_ref[...] = xt[:, 0:64]
    q1_ref[...] = xt[:, 64:128]
    q2_ref[...] = xt[:, 128:192]
    q3_ref[...] = xt[:, 192:256]


def _layer2_body(a0_ref, a1_ref, a2_ref, a3_ref, x0_ref, x1_ref, x2_ref,
                 x3_ref, deg_ref, w_ref, b_ref, x_out_ref):
    blk = x_out_ref.shape[0]
    m = _row_mask(blk)
    dinv = jnp.where(m, _dinv_of(deg_ref[...]), 0.0)
    px = jnp.concatenate(
        [a0_ref[...] + x0_ref[...], a1_ref[...] + x1_ref[...],
         a2_ref[...] + x2_ref[...], a3_ref[...] + x3_ref[...]],
        axis=1) * dinv
    h = jnp.dot(px, w_ref[...], preferred_element_type=jnp.float32) + b_ref[...]
    x_out_ref[...] = jnp.maximum(h, 0.0)


def _dot_t(x, w):
    return lax.dot_general(x, w, (((1,), (1,)), ((), ())),
                           preferred_element_type=jnp.float32)


def _jk_body(x1_ref, x2_ref, deg_ref, wihf_ref, whhf_ref, bf_ref,
             wihb_ref, whhb_ref, bb_ref, wf_ref, wb_ref, attb_ref,
             linw_ref, lo_ref, hi_ref):
    blk = lo_ref.shape[0]
    x1 = x1_ref[...]
    x2 = x2_ref[...]
    m = _row_mask(blk)
    dinv = jnp.where(m, _dinv_of(deg_ref[...]), 0.0)

    bf = bf_ref[...]
    bb = bb_ref[...]

    g0 = _dot_t(x1, wihf_ref[...]) + bf
    i0 = jax.nn.sigmoid(g0[:, 0:256])
    c1 = i0 * jnp.tanh(g0[:, 512:768])
    o0 = jax.nn.sigmoid(g0[:, 768:1024])
    h1f = o0 * jnp.tanh(c1)

    g1 = _dot_t(x2, wihf_ref[...]) + _dot_t(h1f, whhf_ref[...]) + bf
    i1 = jax.nn.sigmoid(g1[:, 0:256])
    f1 = jax.nn.sigmoid(g1[:, 256:512])
    c2 = f1 * c1 + i1 * jnp.tanh(g1[:, 512:768])
    o1 = jax.nn.sigmoid(g1[:, 768:1024])
    h2f = o1 * jnp.tanh(c2)

    gb0 = _dot_t(x2, wihb_ref[...]) + bb
    ib0 = jax.nn.sigmoid(gb0[:, 0:256])
    cb1 = ib0 * jnp.tanh(gb0[:, 512:768])
    ob0 = jax.nn.sigmoid(gb0[:, 768:1024])
    h1b = ob0 * jnp.tanh(cb1)

    gb1 = _dot_t(x1, wihb_ref[...]) + _dot_t(h1b, whhb_ref[...]) + bb
    ib1 = jax.nn.sigmoid(gb1[:, 0:256])
    fb1 = jax.nn.sigmoid(gb1[:, 256:512])
    cb2 = fb1 * cb1 + ib1 * jnp.tanh(gb1[:, 512:768])
    ob1 = jax.nn.sigmoid(gb1[:, 768:1024])
    h2b = ob1 * jnp.tanh(cb2)

    wf = wf_ref[...]
    wb = wb_ref[...]
    attb = attb_ref[...]
    s0 = (jnp.sum(h1f * wf, axis=1, keepdims=True)
          + jnp.sum(h2b * wb, axis=1, keepdims=True) + attb)
    s1 = (jnp.sum(h2f * wf, axis=1, keepdims=True)
          + jnp.sum(h1b * wb, axis=1, keepdims=True) + attb)
    mx = jnp.maximum(s0, s1)
    e0 = jnp.exp(s0 - mx)
    e1 = jnp.exp(s1 - mx)
    a0 = e0 / (e0 + e1)
    a1 = e1 / (e0 + e1)
    xjk = a0 * x1 + a1 * x2

    g = jnp.dot(xjk, linw_ref[...], preferred_element_type=jnp.float32)
    gt = g * dinv
    lo_ref[...] = gt[:, :32]
    hi_ref[...] = gt[:, 32:]


def _final_body(a0_ref, a1_ref, g0_ref, g1_ref, deg_ref, b_ref, o_ref):
    dinv = _dinv_of(deg_ref[...])
    logits = jnp.concatenate(
        [a0_ref[...] + g0_ref[...], a1_ref[...] + g1_ref[...]],
        axis=1) * dinv + b_ref[...]
    mx = jnp.max(logits, axis=1, keepdims=True)
    ex = jnp.exp(logits - mx)
    lse = jnp.log(jnp.sum(ex, axis=1, keepdims=True)) + mx
    o_ref[...] = logits - lse


def _row_spec(blk, width):
    return pl.BlockSpec((blk, width), lambda i: (i, 0))


def _full_spec(shape):
    return pl.BlockSpec(shape, lambda i: tuple(0 for _ in shape))


def kernel(x, edge_index, W1, b1, W2, b2, Wih_f, Whh_f, bih_f, bhh_f,
           Wih_b, Whh_b, bih_b, bhh_b, att_W, att_b, lin_W, lin_b):
    f32 = jnp.float32

    k = jnp.arange(EP - E, dtype=jnp.int32)
    src1d = jnp.concatenate([edge_index[0], N + (k % 16)])
    dst2d = jnp.concatenate([edge_index[1], k % NP]).reshape(EPR, 128)

    ones16 = jnp.concatenate(
        [jnp.ones((N, 16), f32), jnp.zeros((NP - N, 16), f32)])
    z16 = jnp.zeros((NP, 16), f32)
    deg16, _ = _propagate(16, ones16, ones16, src1d, dst2d, z16)

    blk = NP // 16
    xlo, xhi = pl.pallas_call(
        _prescale_body,
        grid=(16,),
        in_specs=[_row_spec(blk, 128), _row_spec(blk, 16)],
        out_specs=(_row_spec(blk, 64), _row_spec(blk, 64)),
        out_shape=(jax.ShapeDtypeStruct((NP, 64), f32),
                   jax.ShapeDtypeStruct((NP, 64), f32)),
    )(x, deg16)

    z64 = jnp.zeros((NP, 64), f32)
    a_lo, a_hi = _propagate(64, xlo, xhi, src1d, dst2d, z64)

    x1, q0, q1, q2, q3 = pl.pallas_call(
        _layer1_body,
        grid=(16,),
        in_specs=[_row_spec(blk, 64), _row_spec(blk, 64),
                  _row_spec(blk, 64), _row_spec(blk, 64),
                  _row_spec(blk, 16),
                  _full_spec((128, 256)), _full_spec((1, 256))],
        out_specs=(_row_spec(blk, 256), _row_spec(blk, 64), _row_spec(blk, 64),
                   _row_spec(blk, 64), _row_spec(blk, 64)),
        out_shape=(jax.ShapeDtypeStruct((NP, 256), f32),)
        + tuple(jax.ShapeDtypeStruct((NP, 64), f32) for _ in range(4)),
    )(a_lo, a_hi, xlo, xhi, deg16, W1, b1.reshape(1, 256))

    b0, b1_ = _propagate(64, q0, q1, src1d, dst2d, z64)
    b2_, b3 = _propagate(64, q2, q3, src1d, dst2d, z64)

    x2 = pl.pallas_call(
        _layer2_body,
        grid=(16,),
        in_specs=[_row_spec(blk, 64)] * 8 + [
            _row_spec(blk, 16), _full_spec((256, 256)), _full_spec((1, 256))],
        out_specs=_row_spec(blk, 256),
        out_shape=jax.ShapeDtypeStruct((NP, 256), f32),
    )(b0, b1_, b2_, b3, q0, q1, q2, q3, deg16, W2, b2.reshape(1, 256))

    jblk = NP // 8
    glo, ghi = pl.pallas_call(
        _jk_body,
        grid=(8,),
        in_specs=[_row_spec(jblk, 256), _row_spec(jblk, 256),
                  _row_spec(jblk, 16),
                  _full_spec((1024, 256)), _full_spec((1024, 256)),
                  _full_spec((1, 1024)),
                  _full_spec((1024, 256)), _full_spec((1024, 256)),
                  _full_spec((1, 1024)),
                  _full_spec((1, 256)), _full_spec((1, 256)),
                  _full_spec((1, 1)), _full_spec((256, 64))],
        out_specs=(_row_spec(jblk, 32), _row_spec(jblk, 32)),
        out_shape=(jax.ShapeDtypeStruct((NP, 32), f32),
                   jax.ShapeDtypeStruct((NP, 32), f32)),
    )(x1, x2, deg16,
      Wih_f, Whh_f, (bih_f + bhh_f).reshape(1, 1024),
      Wih_b, Whh_b, (bih_b + bhh_b).reshape(1, 1024),
      att_W[:256].reshape(1, 256), att_W[256:].reshape(1, 256),
      att_b.reshape(1, 1), lin_W)

    z32 = jnp.zeros((NP, 32), f32)
    c_lo, c_hi = _propagate(32, glo, ghi, src1d, dst2d, z32)

    out = pl.pallas_call(
        _final_body,
        grid=(10,),
        in_specs=[_row_spec(1000, 32), _row_spec(1000, 32),
                  _row_spec(1000, 32), _row_spec(1000, 32),
                  _row_spec(1000, 16), _full_spec((1, 64))],
        out_specs=_row_spec(1000, 64),
        out_shape=jax.ShapeDtypeStruct((N, 64), f32),
    )(c_lo, c_hi, glo, ghi, deg16, lin_b.reshape(1, 64))
    return out


# speedup vs baseline: 23.1017x; 1.0890x over previous
import functools

import jax
import jax.numpy as jnp
from jax import lax
from jax.experimental import pallas as pl
from jax.experimental.pallas import tpu as pltpu
from jax.experimental.pallas import tpu_sc as plsc

N = 10000
E = 320000
NP = 10112
EP = 327680
EPR = EP // 128
NTILES = 16
RPT = NP // NTILES
_NB = 4


@functools.lru_cache(maxsize=None)
def _make_propagate(fh, edge_split):
    mesh = plsc.VectorSubcoreMesh(core_axis_name="c", subcore_axis_name="s",
                                  num_cores=2, num_subcores=16)
    nwork = 32 if edge_split else 16
    ept = EP // nwork
    wpt = ept // 128

    @functools.partial(
        pl.kernel,
        out_type=(
            jax.ShapeDtypeStruct((NP, fh), jnp.float32),
            jax.ShapeDtypeStruct((NP, fh), jnp.float32),
        ),
        mesh=mesh,
        scratch_types=[
            pltpu.VMEM((ept,), jnp.int32),
            pltpu.VMEM((wpt, 128), jnp.int32),
            [pltpu.VMEM((128, fh), jnp.float32) for _ in range(_NB)],
            pltpu.VMEM_SHARED((NP, fh), jnp.float32),
            [pltpu.SemaphoreType.DMA for _ in range(_NB)],
        ],
        compiler_params=pltpu.CompilerParams(use_tc_tiling_on_sc=False),
    )
    def prop(h_lo, h_hi, src1d, dst2d, zeros_hbm, out_lo, out_hi,
             src_v, dst_v, rows, acc, sems):
        c = lax.axis_index("c")
        s = lax.axis_index("s")
        wid = c * NTILES + s if edge_split else s

        e0 = pl.multiple_of(wid * ept, ept)
        pltpu.sync_copy(src1d.at[pl.ds(e0, ept)], src_v)
        w0 = pl.multiple_of(wid * wpt, wpt)
        pltpu.sync_copy(dst2d.at[pl.ds(w0, wpt)], dst_v)

        r0 = pl.multiple_of(s * RPT, RPT)
        pltpu.sync_copy(zeros_hbm.at[pl.ds(r0, RPT)], acc.at[pl.ds(r0, RPT)])
        plsc.subcore_barrier()

        def start_gather(w, b):
            i0 = pl.multiple_of(w * 128, 128)
            idx = src_v.at[pl.ds(i0, 128)]
            if edge_split:
                pltpu.async_copy(h_lo.at[idx], rows[b], sems[b])
            else:
                @pl.when(c == 0)
                def _():
                    pltpu.async_copy(h_lo.at[idx], rows[b], sems[b])

                @pl.when(c != 0)
                def _():
                    pltpu.async_copy(h_hi.at[idx], rows[b], sems[b])

        def wait_gather(b):
            pltpu.make_async_copy(h_lo.at[src_v.at[pl.ds(0, 128)]],
                                  rows[b], sems[b]).wait()

        for b in range(_NB):
            start_gather(b, b)

        def body(g, carry):
            for b in range(_NB):
                w = g * _NB + b
                wait_gather(b)
                pltpu.sync_copy(rows[b], acc.at[dst_v.at[w]], add=True)
                start_gather(w + _NB, b)
            return carry

        lax.fori_loop(0, wpt // _NB - 1, body, 0)
        for b in range(_NB):
            w = (wpt // _NB - 1) * _NB + b
            wait_gather(b)
            pltpu.sync_copy(rows[b], acc.at[dst_v.at[w]], add=True)
        plsc.subcore_barrier()

        @pl.when(c == 0)
        def _():
            pltpu.sync_copy(acc.at[pl.ds(r0, RPT)], out_lo.at[pl.ds(r0, RPT)])

        @pl.when(c != 0)
        def _():
            pltpu.sync_copy(acc.at[pl.ds(r0, RPT)], out_hi.at[pl.ds(r0, RPT)])

    return prop


def _propagate(fh, edge_split, h_lo, h_hi, src1d, dst2d, zeros_hbm):
    return _make_propagate(fh, edge_split)(h_lo, h_hi, src1d, dst2d, zeros_hbm)


def _row_mask(blk):
    base = pl.program_id(0) * blk
    rid = lax.broadcasted_iota(jnp.int32, (blk, 1), 0) + base
    return rid < N


def _prescale_body(x_ref, dega_ref, degb_ref, lo_ref, hi_ref, dinv_ref):
    blk = lo_ref.shape[0]
    m = _row_mask(blk)
    dinv = lax.rsqrt(dega_ref[:, 0:1] + degb_ref[:, 0:1] + 1.0)
    dinv_ref[...] = jnp.broadcast_to(dinv, dinv_ref.shape)
    xt = jnp.where(m, x_ref[...] * dinv, 0.0)
    lo_ref[...] = xt[:, :64]
    hi_ref[...] = xt[:, 64:]


def _layer1_body(a0_ref, a1_ref, x0_ref, x1_ref, dinv_ref, w_ref, b_ref,
                 x_out_ref, q0_ref, q1_ref, q2_ref, q3_ref):
    blk = x_out_ref.shape[0]
    m = _row_mask(blk)
    dinv = jnp.where(m, dinv_ref[:, 0:1], 0.0)
    px = jnp.concatenate(
        [a0_ref[...] + x0_ref[...], a1_ref[...] + x1_ref[...]], axis=1) * dinv
    h = jnp.dot(px, w_ref[...], preferred_element_type=jnp.float32) + b_ref[...]
    xo = jnp.maximum(h, 0.0)
    x_out_ref[...] = xo
    xt = xo * dinv
    q0_ref[...] = xt[:, 0:64]
    q1_ref[...] = xt[:, 64:128]
    q2_ref[...] = xt[:, 128:192]
    q3_ref[...] = xt[:, 192:256]


def _layer2_body(a0_ref, a1_ref, a2_ref, a3_ref, x0_ref, x1_ref, x2_ref,
                 x3_ref, dinv_ref, w_ref, b_ref, x_out_ref):
    blk = x_out_ref.shape[0]
    m = _row_mask(blk)
    dinv = jnp.where(m, dinv_ref[:, 0:1], 0.0)
    px = jnp.concatenate(
        [a0_ref[...] + x0_ref[...], a1_ref[...] + x1_ref[...],
         a2_ref[...] + x2_ref[...], a3_ref[...] + x3_ref[...]],
        axis=1) * dinv
    h = jnp.dot(px, w_ref[...], preferred_element_type=jnp.float32) + b_ref[...]
    x_out_ref[...] = jnp.maximum(h, 0.0)


def _dot_t(x, w):
    return lax.dot_general(x, w, (((1,), (1,)), ((), ())),
                           preferred_element_type=jnp.float32)


def _jk_body(x1_ref, x2_ref, dinv_ref, wihf_ref, whhf_ref, bf_ref,
             wihb_ref, whhb_ref, bb_ref, wf_ref, wb_ref, attb_ref,
             linw_ref, gt_ref):
    blk = gt_ref.shape[0]
    x1 = x1_ref[...]
    x2 = x2_ref[...]
    m = _row_mask(blk)
    dinv = jnp.where(m, dinv_ref[:, 0:1], 0.0)

    bf = bf_ref[...]
    bb = bb_ref[...]

    g0 = _dot_t(x1, wihf_ref[...]) + bf
    i0 = jax.nn.sigmoid(g0[:, 0:256])
    c1 = i0 * jnp.tanh(g0[:, 512:768])
    o0 = jax.nn.sigmoid(g0[:, 768:1024])
    h1f = o0 * jnp.tanh(c1)

    g1 = _dot_t(x2, wihf_ref[...]) + _dot_t(h1f, whhf_ref[...]) + bf
    i1 = jax.nn.sigmoid(g1[:, 0:256])
    f1 = jax.nn.sigmoid(g1[:, 256:512])
    c2 = f1 * c1 + i1 * jnp.tanh(g1[:, 512:768])
    o1 = jax.nn.sigmoid(g1[:, 768:1024])
    h2f = o1 * jnp.tanh(c2)

    gb0 = _dot_t(x2, wihb_ref[...]) + bb
    ib0 = jax.nn.sigmoid(gb0[:, 0:256])
    cb1 = ib0 * jnp.tanh(gb0[:, 512:768])
    ob0 = jax.nn.sigmoid(gb0[:, 768:1024])
    h1b = ob0 * jnp.tanh(cb1)

    gb1 = _dot_t(x1, wihb_ref[...]) + _dot_t(h1b, whhb_ref[...]) + bb
    ib1 = jax.nn.sigmoid(gb1[:, 0:256])
    fb1 = jax.nn.sigmoid(gb1[:, 256:512])
    cb2 = fb1 * cb1 + ib1 * jnp.tanh(gb1[:, 512:768])
    ob1 = jax.nn.sigmoid(gb1[:, 768:1024])
    h2b = ob1 * jnp.tanh(cb2)

    wf = wf_ref[...]
    wb = wb_ref[...]
    attb = attb_ref[...]
    s0 = (jnp.sum(h1f * wf, axis=1, keepdims=True)
          + jnp.sum(h2b * wb, axis=1, keepdims=True) + attb)
    s1 = (jnp.sum(h2f * wf, axis=1, keepdims=True)
          + jnp.sum(h1b * wb, axis=1, keepdims=True) + attb)
    mx = jnp.maximum(s0, s1)
    e0 = jnp.exp(s0 - mx)
    e1 = jnp.exp(s1 - mx)
    a0 = e0 / (e0 + e1)
    a1 = e1 / (e0 + e1)
    xjk = a0 * x1 + a1 * x2

    g = jnp.dot(xjk, linw_ref[...], preferred_element_type=jnp.float32)
    gt_ref[...] = g * dinv


def _final_body(ca_ref, cb_ref, gt_ref, dinv_ref, b_ref, o_ref):
    dinv = dinv_ref[:, 0:1]
    logits = (ca_ref[...] + cb_ref[...] + gt_ref[...]) * dinv + b_ref[...]
    mx = jnp.max(logits, axis=1, keepdims=True)
    ex = jnp.exp(logits - mx)
    lse = jnp.log(jnp.sum(ex, axis=1, keepdims=True)) + mx
    o_ref[...] = logits - lse


def _row_spec(blk, width):
    return pl.BlockSpec((blk, width), lambda i: (i, 0))


def _full_spec(shape):
    return pl.BlockSpec(shape, lambda i: tuple(0 for _ in shape))


def kernel(x, edge_index, W1, b1, W2, b2, Wih_f, Whh_f, bih_f, bhh_f,
           Wih_b, Whh_b, bih_b, bhh_b, att_W, att_b, lin_W, lin_b):
    f32 = jnp.float32

    k = jnp.arange(EP - E, dtype=jnp.int32)
    src1d = jnp.concatenate([edge_index[0], N + (k % 16)])
    dst2d = jnp.concatenate([edge_index[1], k % NP]).reshape(EPR, 128)

    ones16 = jnp.concatenate(
        [jnp.ones((N, 16), f32), jnp.zeros((NP - N, 16), f32)])
    z16 = jnp.zeros((NP, 16), f32)
    dega, degb = _propagate(16, True, ones16, ones16, src1d, dst2d, z16)

    blk = NP // 8
    xlo, xhi, dinv16 = pl.pallas_call(
        _prescale_body,
        grid=(8,),
        in_specs=[_row_spec(blk, 128), _row_spec(blk, 16), _row_spec(blk, 16)],
        out_specs=(_row_spec(blk, 64), _row_spec(blk, 64), _row_spec(blk, 16)),
        out_shape=(jax.ShapeDtypeStruct((NP, 64), f32),
                   jax.ShapeDtypeStruct((NP, 64), f32),
                   jax.ShapeDtypeStruct((NP, 16), f32)),
    )(x, dega, degb)

    z64 = jnp.zeros((NP, 64), f32)
    a_lo, a_hi = _propagate(64, False, xlo, xhi, src1d, dst2d, z64)

    x1, q0, q1, q2, q3 = pl.pallas_call(
        _layer1_body,
        grid=(8,),
        in_specs=[_row_spec(blk, 64), _row_spec(blk, 64),
                  _row_spec(blk, 64), _row_spec(blk, 64),
                  _row_spec(blk, 16),
                  _full_spec((128, 256)), _full_spec((1, 256))],
        out_specs=(_row_spec(blk, 256), _row_spec(blk, 64), _row_spec(blk, 64),
                   _row_spec(blk, 64), _row_spec(blk, 64)),
        out_shape=(jax.ShapeDtypeStruct((NP, 256), f32),)
        + tuple(jax.ShapeDtypeStruct((NP, 64), f32) for _ in range(4)),
    )(a_lo, a_hi, xlo, xhi, dinv16, W1, b1.reshape(1, 256))

    b0, b1_ = _propagate(64, False, q0, q1, src1d, dst2d, z64)
    b2_, b3 = _propagate(64, False, q2, q3, src1d, dst2d, z64)

    x2 = pl.pallas_call(
        _layer2_body,
        grid=(8,),
        in_specs=[_row_spec(blk, 64)] * 8 + [
            _row_spec(blk, 16), _full_spec((256, 256)), _full_spec((1, 256))],
        out_specs=_row_spec(blk, 256),
        out_shape=jax.ShapeDtypeStruct((NP, 256), f32),
    )(b0, b1_, b2_, b3, q0, q1, q2, q3, dinv16, W2, b2.reshape(1, 256))

    gt = pl.pallas_call(
        _jk_body,
        grid=(8,),
        in_specs=[_row_spec(blk, 256), _row_spec(blk, 256),
                  _row_spec(blk, 16),
                  _full_spec((1024, 256)), _full_spec((1024, 256)),
                  _full_spec((1, 1024)),
                  _full_spec((1024, 256)), _full_spec((1024, 256)),
                  _full_spec((1, 1024)),
                  _full_spec((1, 256)), _full_spec((1, 256)),
                  _full_spec((1, 1)), _full_spec((256, 64))],
        out_specs=_row_spec(blk, 64),
        out_shape=jax.ShapeDtypeStruct((NP, 64), f32),
    )(x1, x2, dinv16,
      Wih_f, Whh_f, (bih_f + bhh_f).reshape(1, 1024),
      Wih_b, Whh_b, (bih_b + bhh_b).reshape(1, 1024),
      att_W[:256].reshape(1, 256), att_W[256:].reshape(1, 256),
      att_b.reshape(1, 1), lin_W)

    z64b = jnp.zeros((NP, 64), f32)
    ca, cb = _propagate(64, True, gt, gt, src1d, dst2d, z64b)

    out = pl.pallas_call(
        _final_body,
        grid=(10,),
        in_specs=[_row_spec(1000, 64), _row_spec(1000, 64),
                  _row_spec(1000, 64), _row_spec(1000, 16),
                  _full_spec((1, 64))],
        out_specs=_row_spec(1000, 64),
        out_shape=jax.ShapeDtypeStruct((N, 64), f32),
    )(ca, cb, gt, dinv16, lin_b.reshape(1, 64))
    return out
